# Initial kernel scaffold; baseline (speedup 1.0000x reference)
import jax
import jax.numpy as jnp
from jax.experimental import pallas as pl


def kernel(latent_grid, coords, params, W1, b1, W2, b2, W3, b3):
    raise NotImplementedError("write your pallas kernel here")


# speedup vs baseline: 6.8037x; 6.8037x over previous
import functools

import jax
import jax.numpy as jnp
from jax import lax
from jax.experimental import pallas as pl
from jax.experimental.pallas import tpu as pltpu
from jax.experimental.pallas import tpu_sc as plsc

Q_N = 2048
K_N = 100000
DP = 64
KNN = 16
H_N = 128

QB = 256
KC = 4096
NCHUNK = 25
K_PAD = NCHUNK * KC
NTILE = 32
QPT = Q_N // NTILE
FW = 128
BIG = 1e30
IMAX = jnp.iinfo(jnp.int32).max


def _knn_block(lg_ref, ct_ref, c2_ref, idx_ref, bestv, besti):
    c = pl.program_id(1)

    @pl.when(c == 0)
    def _init():
        bestv[...] = jnp.full((QB, KNN), BIG, jnp.float32)
        besti[...] = jnp.zeros((QB, KNN), jnp.int32)

    lg = lg_ref[...]
    qc = jnp.dot(lg, ct_ref[...], preferred_element_type=jnp.float32)
    t = c2_ref[...] - 2.0 * qc

    base = c * KC
    lane = lax.broadcasted_iota(jnp.int32, (QB, 128), 1)
    m1 = t[:, 0:128]
    i1 = lane + base
    m2 = jnp.full((QB, 128), BIG, jnp.float32)
    i2 = jnp.zeros((QB, 128), jnp.int32)
    for cc in range(1, KC // 128):
        x = t[:, cc * 128:(cc + 1) * 128]
        g = lane + (base + cc * 128)
        lt = x < m1
        hi = jnp.where(lt, m1, x)
        hi_i = jnp.where(lt, i1, g)
        m1 = jnp.where(lt, x, m1)
        i1 = jnp.where(lt, g, i1)
        lt2 = hi < m2
        m2 = jnp.where(lt2, hi, m2)
        i2 = jnp.where(lt2, hi_i, i2)

    work = jnp.concatenate([bestv[...], m1, m2], axis=1)
    gidx = jnp.concatenate([besti[...], i1, i2], axis=1)
    for s in range(KNN):
        m = jnp.min(work, axis=1, keepdims=True)
        sel = jnp.min(jnp.where(work == m, gidx, IMAX), axis=1,
                      keepdims=True)
        bestv[:, s:s + 1] = m
        besti[:, s:s + 1] = sel
        work = jnp.where(gidx == sel, BIG, work)

    @pl.when(c == NCHUNK - 1)
    def _emit():
        idx_ref[...] = besti[...]


def _knn_topk(lg, ctp, c2p):
    return pl.pallas_call(
        _knn_block,
        grid=(Q_N // QB, NCHUNK),
        in_specs=[
            pl.BlockSpec((QB, 3), lambda q, c: (q, 0)),
            pl.BlockSpec((3, KC), lambda q, c: (0, c)),
            pl.BlockSpec((1, KC), lambda q, c: (0, c)),
        ],
        out_specs=pl.BlockSpec((QB, KNN), lambda q, c: (q, 0)),
        out_shape=jax.ShapeDtypeStruct((Q_N, KNN), jnp.int32),
        scratch_shapes=[
            pltpu.VMEM((QB, KNN), jnp.float32),
            pltpu.VMEM((QB, KNN), jnp.int32),
        ],
    )(lg, ctp, c2p)


def _gather_body(idx_hbm, ctab_hbm, out_hbm, idxv, rows, obuf, semg):
    wid = lax.axis_index("s") * 2 + lax.axis_index("c")
    q0 = wid * QPT
    pltpu.sync_copy(idx_hbm.at[pl.ds(q0 * KNN, QPT * KNN)], idxv)
    for half in range(2):
        nrow = QPT * KNN // 2
        pltpu.async_copy(
            ctab_hbm.at[idxv.at[pl.ds(half * nrow, nrow)]], rows, semg
        ).wait()

        def q_body(i, _):
            for cv in range(FW // 16):
                a0 = rows[i * KNN, pl.ds(cv * 16, 16)]

                def nb(j, a):
                    return a + rows[i * KNN + j, pl.ds(cv * 16, 16)]
                acc = lax.fori_loop(1, KNN, nb, a0)
                obuf[pl.ds((half * (QPT // 2) + i) * FW + cv * 16, 16)] = (
                    acc * (1.0 / KNN))
            return 0
        lax.fori_loop(0, QPT // 2, q_body, 0)
    pltpu.sync_copy(obuf, out_hbm.at[pl.ds(q0 * FW, QPT * FW)])


def _gather_mean(idx_flat, ctab):
    mesh = plsc.VectorSubcoreMesh(core_axis_name="c", subcore_axis_name="s")
    f = functools.partial(
        pl.kernel,
        out_type=jax.ShapeDtypeStruct((Q_N * FW,), jnp.float32),
        mesh=mesh,
        scratch_types=[
            pltpu.VMEM((QPT * KNN,), jnp.int32),
            pltpu.VMEM((QPT * KNN // 2, FW), jnp.float32),
            pltpu.VMEM((QPT * FW,), jnp.float32),
            pltpu.SemaphoreType.DMA,
        ],
    )(_gather_body)
    return f(idx_flat, ctab)


def _mlp_block(x_ref, lg_ref, w1_ref, w1q_ref, b1_ref, w2_ref, b2_ref,
               w3_ref, b3_ref, o_ref):
    h = jnp.dot(x_ref[...], w1_ref[...], preferred_element_type=jnp.float32)
    h = h + jnp.dot(lg_ref[...], w1q_ref[...],
                    preferred_element_type=jnp.float32)
    h = jnp.maximum(h + b1_ref[...], 0.0)
    h = jnp.dot(h, w2_ref[...], preferred_element_type=jnp.float32)
    h = jnp.maximum(h + b2_ref[...], 0.0)
    o_ref[...] = (jnp.dot(h, w3_ref[...], preferred_element_type=jnp.float32)
                  + b3_ref[...])


def _mlp(x, lg, w1p, w1q, b1, w2, b2, w3, b3):
    blk = 512
    return pl.pallas_call(
        _mlp_block,
        grid=(Q_N // blk,),
        in_specs=[
            pl.BlockSpec((blk, FW), lambda i: (i, 0)),
            pl.BlockSpec((blk, 3), lambda i: (i, 0)),
            pl.BlockSpec((FW, H_N), lambda i: (0, 0)),
            pl.BlockSpec((3, H_N), lambda i: (0, 0)),
            pl.BlockSpec((1, H_N), lambda i: (0, 0)),
            pl.BlockSpec((H_N, H_N), lambda i: (0, 0)),
            pl.BlockSpec((1, H_N), lambda i: (0, 0)),
            pl.BlockSpec((H_N, H_N), lambda i: (0, 0)),
            pl.BlockSpec((1, H_N), lambda i: (0, 0)),
        ],
        out_specs=pl.BlockSpec((blk, H_N), lambda i: (i, 0)),
        out_shape=jax.ShapeDtypeStruct((Q_N, H_N), jnp.float32),
    )(x, lg, w1p, w1q, b1, w2, b2, w3, b3)


def kernel(latent_grid, coords, params, W1, b1, W2, b2, W3, b3):
    f32 = jnp.float32
    padk = K_PAD - K_N
    ctp = jnp.concatenate([coords.T, jnp.zeros((3, padk), f32)], axis=1)
    c2p = jnp.concatenate([jnp.sum(coords * coords, axis=1),
                           jnp.full((padk,), BIG, f32)]).reshape(1, K_PAD)
    ctab = jnp.concatenate(
        [params, coords, jnp.zeros((K_N, FW - DP - 3), f32)], axis=1)
    w1p = jnp.concatenate(
        [W1[6:], W1[3:6], jnp.zeros((FW - W1.shape[0] + 3, H_N), f32)],
        axis=0)
    w1q = W1[:3]

    idx = _knn_topk(latent_grid, ctp, c2p)
    enc = _gather_mean(idx.reshape(-1), ctab).reshape(Q_N, FW)
    return _mlp(enc, latent_grid, w1p, w1q, b1.reshape(1, -1),
                W2, b2.reshape(1, -1), W3, b3.reshape(1, -1))


# speedup vs baseline: 9.1904x; 1.3508x over previous
import functools

import jax
import jax.numpy as jnp
from jax import lax
from jax.experimental import pallas as pl
from jax.experimental.pallas import tpu as pltpu
from jax.experimental.pallas import tpu_sc as plsc

Q_N = 2048
K_N = 100000
DP = 64
KNN = 16
H_N = 128

QB = 256
KC = 8192
NCHUNK = 13
K_PAD = NCHUNK * KC
NTILE = 32
QPT = Q_N // NTILE
FW = 128
BIG = 1e30
IMAX = jnp.iinfo(jnp.int32).max


def _knn_block(lg_ref, ct_ref, c2_ref, idx_ref, bestv, besti):
    c = pl.program_id(1)

    @pl.when(c == 0)
    def _init():
        bestv[...] = jnp.full((QB, KNN), BIG, jnp.float32)
        besti[...] = jnp.zeros((QB, KNN), jnp.float32)

    lg = lg_ref[...]
    qc = jnp.dot(lg, ct_ref[...], preferred_element_type=jnp.float32)
    t = c2_ref[...] - 2.0 * qc

    lane = lax.broadcasted_iota(jnp.int32, (QB, 128), 1).astype(jnp.float32)
    base_f = (c * KC).astype(jnp.float32)
    m1 = t[:, 0:128]
    i1 = lane + base_f
    m2 = jnp.full((QB, 128), BIG, jnp.float32)
    i2 = jnp.full((QB, 128), -1.0, jnp.float32)
    m3 = jnp.full((QB, 128), BIG, jnp.float32)
    i3 = jnp.full((QB, 128), -1.0, jnp.float32)
    for cc in range(1, KC // 128):
        x = t[:, cc * 128:(cc + 1) * 128]
        g = lane + (base_f + float(cc * 128))
        lt1 = x < m1
        hi1 = jnp.maximum(m1, x)
        hi1_i = jnp.where(lt1, i1, g)
        m1 = jnp.minimum(m1, x)
        i1 = jnp.where(lt1, g, i1)
        lt2 = hi1 < m2
        hi2 = jnp.maximum(m2, hi1)
        hi2_i = jnp.where(lt2, i2, hi1_i)
        m2 = jnp.minimum(m2, hi1)
        i2 = jnp.where(lt2, hi1_i, i2)
        lt3 = hi2 < m3
        m3 = jnp.minimum(m3, hi2)
        i3 = jnp.where(lt3, hi2_i, i3)

    bigq = jnp.full((QB, KNN), BIG, jnp.float32)
    noneq = jnp.full((QB, KNN), -1.0, jnp.float32)
    work = jnp.concatenate([bestv[...], m1], axis=1)
    gidx = jnp.concatenate([besti[...], i1], axis=1)
    repv1 = jnp.concatenate([bigq, m2], axis=1)
    repi1 = jnp.concatenate([noneq, i2], axis=1)
    repv2 = jnp.concatenate([bigq, m3], axis=1)
    repi2 = jnp.concatenate([noneq, i3], axis=1)
    for s in range(KNN):
        m = jnp.min(work, axis=1, keepdims=True)
        sel = jnp.min(jnp.where(work == m, gidx, 3e8), axis=1,
                      keepdims=True)
        bestv[:, s:s + 1] = m
        besti[:, s:s + 1] = sel
        hit = gidx == sel
        work = jnp.where(hit, repv1, work)
        gidx = jnp.where(hit, repi1, gidx)
        repv1 = jnp.where(hit, repv2, repv1)
        repi1 = jnp.where(hit, repi2, repi1)
        repv2 = jnp.where(hit, BIG, repv2)

    @pl.when(c == NCHUNK - 1)
    def _emit():
        idx_ref[...] = besti[...].astype(jnp.int32)


def _knn_topk(lg, ctp, c2p):
    return pl.pallas_call(
        _knn_block,
        grid=(Q_N // QB, NCHUNK),
        in_specs=[
            pl.BlockSpec((QB, 3), lambda q, c: (q, 0)),
            pl.BlockSpec((3, KC), lambda q, c: (0, c)),
            pl.BlockSpec((1, KC), lambda q, c: (0, c)),
        ],
        out_specs=pl.BlockSpec((QB, KNN), lambda q, c: (q, 0)),
        out_shape=jax.ShapeDtypeStruct((Q_N, KNN), jnp.int32),
        scratch_shapes=[
            pltpu.VMEM((QB, KNN), jnp.float32),
            pltpu.VMEM((QB, KNN), jnp.float32),
        ],
    )(lg, ctp, c2p)


def _gather_body(idx_hbm, ctab_hbm, out_hbm, idxv, rows, obuf, semg):
    wid = lax.axis_index("s") * 2 + lax.axis_index("c")
    q0 = wid * QPT
    pltpu.sync_copy(idx_hbm.at[pl.ds(q0 * KNN, QPT * KNN)], idxv)
    for half in range(2):
        nrow = QPT * KNN // 2
        pltpu.async_copy(
            ctab_hbm.at[idxv.at[pl.ds(half * nrow, nrow)]], rows, semg
        ).wait()

        def q_body(i, _):
            for cv in range(FW // 16):
                a0 = rows[i * KNN, pl.ds(cv * 16, 16)]

                def nb(j, a):
                    return a + rows[i * KNN + j, pl.ds(cv * 16, 16)]
                acc = lax.fori_loop(1, KNN, nb, a0)
                obuf[pl.ds((half * (QPT // 2) + i) * FW + cv * 16, 16)] = (
                    acc * (1.0 / KNN))
            return 0
        lax.fori_loop(0, QPT // 2, q_body, 0)
    pltpu.sync_copy(obuf, out_hbm.at[pl.ds(q0 * FW, QPT * FW)])


def _gather_mean(idx_flat, ctab):
    mesh = plsc.VectorSubcoreMesh(core_axis_name="c", subcore_axis_name="s")
    f = functools.partial(
        pl.kernel,
        out_type=jax.ShapeDtypeStruct((Q_N * FW,), jnp.float32),
        mesh=mesh,
        scratch_types=[
            pltpu.VMEM((QPT * KNN,), jnp.int32),
            pltpu.VMEM((QPT * KNN // 2, FW), jnp.float32),
            pltpu.VMEM((QPT * FW,), jnp.float32),
            pltpu.SemaphoreType.DMA,
        ],
    )(_gather_body)
    return f(idx_flat, ctab)


def _mlp_block(x_ref, lg_ref, w1_ref, w1q_ref, b1_ref, w2_ref, b2_ref,
               w3_ref, b3_ref, o_ref):
    h = jnp.dot(x_ref[...], w1_ref[...], preferred_element_type=jnp.float32)
    h = h + jnp.dot(lg_ref[...], w1q_ref[...],
                    preferred_element_type=jnp.float32)
    h = jnp.maximum(h + b1_ref[...], 0.0)
    h = jnp.dot(h, w2_ref[...], preferred_element_type=jnp.float32)
    h = jnp.maximum(h + b2_ref[...], 0.0)
    o_ref[...] = (jnp.dot(h, w3_ref[...], preferred_element_type=jnp.float32)
                  + b3_ref[...])


def _mlp(x, lg, w1p, w1q, b1, w2, b2, w3, b3):
    blk = 512
    return pl.pallas_call(
        _mlp_block,
        grid=(Q_N // blk,),
        in_specs=[
            pl.BlockSpec((blk, FW), lambda i: (i, 0)),
            pl.BlockSpec((blk, 3), lambda i: (i, 0)),
            pl.BlockSpec((FW, H_N), lambda i: (0, 0)),
            pl.BlockSpec((3, H_N), lambda i: (0, 0)),
            pl.BlockSpec((1, H_N), lambda i: (0, 0)),
            pl.BlockSpec((H_N, H_N), lambda i: (0, 0)),
            pl.BlockSpec((1, H_N), lambda i: (0, 0)),
            pl.BlockSpec((H_N, H_N), lambda i: (0, 0)),
            pl.BlockSpec((1, H_N), lambda i: (0, 0)),
        ],
        out_specs=pl.BlockSpec((blk, H_N), lambda i: (i, 0)),
        out_shape=jax.ShapeDtypeStruct((Q_N, H_N), jnp.float32),
    )(x, lg, w1p, w1q, b1, w2, b2, w3, b3)


def kernel(latent_grid, coords, params, W1, b1, W2, b2, W3, b3):
    f32 = jnp.float32
    padk = K_PAD - K_N
    ctp = jnp.concatenate([coords.T, jnp.zeros((3, padk), f32)], axis=1)
    c2p = jnp.concatenate([jnp.sum(coords * coords, axis=1),
                           jnp.full((padk,), BIG, f32)]).reshape(1, K_PAD)
    ctab = jnp.concatenate(
        [params, coords, jnp.zeros((K_N, FW - DP - 3), f32)], axis=1)
    w1p = jnp.concatenate(
        [W1[6:], W1[3:6], jnp.zeros((FW - W1.shape[0] + 3, H_N), f32)],
        axis=0)
    w1q = W1[:3]

    idx = _knn_topk(latent_grid, ctp, c2p)
    enc = _gather_mean(idx.reshape(-1), ctab).reshape(Q_N, FW)
    return _mlp(enc, latent_grid, w1p, w1q, b1.reshape(1, -1),
                W2, b2.reshape(1, -1), W3, b3.reshape(1, -1))
